# baseline (speedup 1.0000x reference)
import functools

import jax
import jax.numpy as jnp
from jax.experimental import pallas as pl
from jax.experimental.pallas import tpu as pltpu
from jax.experimental.pallas import tpu_sc as plsc

_N = 20000
_PRE = 4096
_POST = 500
_NMS_THRESH = 0.1
_SCORE_THRESH = 0.1
_R = 32
_C = 128


def _sup_tile(rb_ref, colaux_ref, cj):
    def ra(d):
        return rb_ref[d]

    def ca(d):
        return colaux_ref[d, pl.ds(cj, 1), :]

    ix = jnp.maximum(jnp.minimum(ra(3), ca(3)) - jnp.maximum(ra(0), ca(0)), 0.0)
    iy = jnp.maximum(jnp.minimum(ra(4), ca(4)) - jnp.maximum(ra(1), ca(1)), 0.0)
    iz = jnp.maximum(jnp.minimum(ra(5), ca(5)) - jnp.maximum(ra(2), ca(2)), 0.0)
    inter = ix * iy * iz
    union = jnp.maximum(ra(6) + ca(6) - inter, 1e-6)
    return (inter > _NMS_THRESH * union).astype(jnp.float32)


def _nms_body(colaux_ref, rowaux_ref, valid_ref, keep_ref, rb_ref):
    keep_ref[...] = valid_ref[...]
    sub = jax.lax.broadcasted_iota(jnp.int32, (_C, _C), 0)
    lanesq = jax.lax.broadcasted_iota(jnp.int32, (_C, _C), 1)

    def block(bi, carry):
        r0 = bi * _C
        for d in range(7):
            rb_ref[d] = jnp.broadcast_to(
                rowaux_ref[d, pl.ds(r0, _C), :], (_C, _C))
        s_bb = _sup_tile(rb_ref, colaux_ref, bi)
        s_bb = jnp.where(lanesq > sub, s_bb, 0.0)
        kb = keep_ref[pl.ds(bi, 1), :]

        def w_cond(c):
            return c[1]

        def w_body(c):
            x, _ = c
            sup = jnp.dot(x, s_bb, preferred_element_type=jnp.float32)
            nx = jnp.where(sup > 0.0, 0.0, kb)
            return nx, jnp.any(nx != x)

        x, _ = jax.lax.while_loop(w_cond, w_body, (kb, True))
        keep_ref[pl.ds(bi, 1), :] = x

        def chunk4(q, carry2):
            for t in range(4):
                cj = q * 4 + t
                s_bc = _sup_tile(rb_ref, colaux_ref, cj)
                sup = jnp.dot(x, s_bc, preferred_element_type=jnp.float32)
                live = (cj > bi).astype(jnp.float32)
                krow = keep_ref[pl.ds(cj, 1), :]
                keep_ref[pl.ds(cj, 1), :] = jnp.where(
                    sup * live > 0.0, 0.0, krow)
            return carry2

        jax.lax.fori_loop((bi + 1) // 4, _R // 4, chunk4, 0)
        return carry

    jax.lax.fori_loop(0, _R, block, 0)


_NW = 32
_BPW = _PRE // _NW


def _sc_gather(table16, idx):
    mesh = plsc.VectorSubcoreMesh(core_axis_name="c", subcore_axis_name="s")

    @functools.partial(
        pl.kernel, mesh=mesh,
        out_type=jax.ShapeDtypeStruct((_PRE, 128), jnp.float32),
        scratch_types=[
            pltpu.VMEM((_BPW,), jnp.int32),
            pltpu.VMEM((_BPW, 128), jnp.float32),
            pltpu.SemaphoreType.DMA,
        ],
    )
    def k(table_hbm, idx_hbm, out_hbm, idx_v, rows_v, sem):
        wid = jax.lax.axis_index("s") * 2 + jax.lax.axis_index("c")
        base = wid * _BPW
        pltpu.sync_copy(idx_hbm.at[pl.ds(base, _BPW)], idx_v)
        pltpu.async_copy(table_hbm.at[idx_v], rows_v, sem).wait()
        pltpu.sync_copy(rows_v, out_hbm.at[pl.ds(base, _BPW)])

    return k(table16, idx)


def kernel(box_scores, box_preds):
    masked = jnp.where(box_scores > _SCORE_THRESH, box_scores, -1.0)
    _, idx = jax.lax.top_k(masked, _PRE)
    table = jnp.pad(
        jnp.concatenate([box_preds, box_scores[:, None]], axis=1),
        ((0, 0), (0, 120)))
    g = _sc_gather(table, idx)
    s = g[:, 7]
    b = g[:, :7]
    c = b[:, :3]
    d = jnp.abs(b[:, 3:6])
    mn = c - d * 0.5
    mx = c + d * 0.5
    vol = (d[:, 0] * d[:, 1] * d[:, 2])[:, None]
    cat = jnp.concatenate([mn, mx, vol], axis=1).T
    colaux = cat.reshape(7, _R, _C)
    rowaux = cat.reshape(7, _PRE, 1)
    valid = (s > _SCORE_THRESH).astype(jnp.float32).reshape(_R, _C)
    keep_f = pl.pallas_call(
        _nms_body,
        out_shape=jax.ShapeDtypeStruct((_R, _C), jnp.float32),
        scratch_shapes=[pltpu.VMEM((7, _C, _C), jnp.float32)],
    )(colaux, rowaux, valid)
    keep = keep_f.reshape(_PRE) > 0.5
    kept_masked = jnp.where(keep, s, -1.0)
    _, order = jax.lax.top_k(kept_masked, _POST)
    sel_valid = keep[order]
    sel_scores = jnp.where(sel_valid, s[order], 0.0)
    sel_idx = jnp.where(sel_valid, idx[order], -1)
    return sel_scores, sel_idx


# speedup vs baseline: 1.0354x; 1.0354x over previous
import functools

import jax
import jax.numpy as jnp
from jax.experimental import pallas as pl
from jax.experimental.pallas import tpu as pltpu
from jax.experimental.pallas import tpu_sc as plsc

_N = 20000
_PRE = 4096
_POST = 500
_NMS_THRESH = 0.1
_SCORE_THRESH = 0.1
_R = 32
_C = 128


def _sup_tile(rb_ref, colaux_ref, cj):
    def ra(d):
        return rb_ref[d]

    def ca(d):
        return colaux_ref[d, pl.ds(cj, 1), :]

    ix = jnp.maximum(jnp.minimum(ra(3), ca(3)) - jnp.maximum(ra(0), ca(0)), 0.0)
    iy = jnp.maximum(jnp.minimum(ra(4), ca(4)) - jnp.maximum(ra(1), ca(1)), 0.0)
    iz = jnp.maximum(jnp.minimum(ra(5), ca(5)) - jnp.maximum(ra(2), ca(2)), 0.0)
    inter = ix * iy * iz
    union = jnp.maximum(ra(6) + ca(6) - inter, 1e-6)
    return (inter > _NMS_THRESH * union).astype(jnp.float32)


def _nms_body(colaux_ref, rowaux_ref, valid_ref, keep_ref, rb_ref):
    keep_ref[...] = valid_ref[...]
    sub = jax.lax.broadcasted_iota(jnp.int32, (_C, _C), 0)
    lanesq = jax.lax.broadcasted_iota(jnp.int32, (_C, _C), 1)

    def block(bi, carry):
        r0 = bi * _C
        for d in range(7):
            rb_ref[d] = jnp.broadcast_to(
                rowaux_ref[d, pl.ds(r0, _C), :], (_C, _C))
        s_bb = _sup_tile(rb_ref, colaux_ref, bi)
        s_bb = jnp.where(lanesq > sub, s_bb, 0.0)
        kb = keep_ref[pl.ds(bi, 1), :]

        def w_cond(c):
            return c[1]

        def w_body(c):
            x, _ = c
            sup = jnp.dot(x, s_bb, preferred_element_type=jnp.float32)
            nx = jnp.where(sup > 0.0, 0.0, kb)
            return nx, jnp.any(nx != x)

        x, _ = jax.lax.while_loop(w_cond, w_body, (kb, True))
        keep_ref[pl.ds(bi, 1), :] = x

        def chunk8(q, carry2):
            for t in range(8):
                cj = q * 8 + t
                s_bc = _sup_tile(rb_ref, colaux_ref, cj)
                sup = jnp.dot(x, s_bc, preferred_element_type=jnp.float32)
                live = (cj > bi).astype(jnp.float32)
                krow = keep_ref[pl.ds(cj, 1), :]
                keep_ref[pl.ds(cj, 1), :] = jnp.where(
                    sup * live > 0.0, 0.0, krow)
            return carry2

        jax.lax.fori_loop((bi + 1) // 8, _R // 8, chunk8, 0)
        return carry

    jax.lax.fori_loop(0, _R, block, 0)


_NW = 32
_BPW = _PRE // _NW


def _sc_gather(table16, idx):
    mesh = plsc.VectorSubcoreMesh(core_axis_name="c", subcore_axis_name="s")

    @functools.partial(
        pl.kernel, mesh=mesh,
        out_type=jax.ShapeDtypeStruct((_PRE, 128), jnp.float32),
        scratch_types=[
            pltpu.VMEM((_BPW,), jnp.int32),
            pltpu.VMEM((_BPW, 128), jnp.float32),
            pltpu.SemaphoreType.DMA,
        ],
    )
    def k(table_hbm, idx_hbm, out_hbm, idx_v, rows_v, sem):
        wid = jax.lax.axis_index("s") * 2 + jax.lax.axis_index("c")
        base = wid * _BPW
        pltpu.sync_copy(idx_hbm.at[pl.ds(base, _BPW)], idx_v)
        pltpu.async_copy(table_hbm.at[idx_v], rows_v, sem).wait()
        pltpu.sync_copy(rows_v, out_hbm.at[pl.ds(base, _BPW)])

    return k(table16, idx)


def kernel(box_scores, box_preds):
    _, idx = jax.lax.top_k(box_scores, _PRE)
    table = jnp.pad(
        jnp.concatenate([box_preds, box_scores[:, None]], axis=1),
        ((0, 0), (0, 120)))
    g = _sc_gather(table, idx)
    s = g[:, 7]
    b = g[:, :7]
    c = b[:, :3]
    d = jnp.abs(b[:, 3:6])
    mn = c - d * 0.5
    mx = c + d * 0.5
    vol = (d[:, 0] * d[:, 1] * d[:, 2])[:, None]
    cat = jnp.concatenate([mn, mx, vol], axis=1).T
    colaux = cat.reshape(7, _R, _C)
    rowaux = cat.reshape(7, _PRE, 1)
    valid = (s > _SCORE_THRESH).astype(jnp.float32).reshape(_R, _C)
    keep_f = pl.pallas_call(
        _nms_body,
        out_shape=jax.ShapeDtypeStruct((_R, _C), jnp.float32),
        scratch_shapes=[pltpu.VMEM((7, _C, _C), jnp.float32)],
    )(colaux, rowaux, valid)
    keep = keep_f.reshape(_PRE) > 0.5
    kept_masked = jnp.where(keep, s, -1.0)
    _, order = jax.lax.top_k(kept_masked, _POST)
    sel_valid = keep[order]
    sel_scores = jnp.where(sel_valid, s[order], 0.0)
    sel_idx = jnp.where(sel_valid, idx[order], -1)
    return sel_scores, sel_idx


# speedup vs baseline: 1.0373x; 1.0019x over previous
import functools

import jax
import jax.numpy as jnp
from jax.experimental import pallas as pl
from jax.experimental.pallas import tpu as pltpu
from jax.experimental.pallas import tpu_sc as plsc

_N = 20000
_PRE = 4096
_POST = 500
_NMS_THRESH = 0.1
_SCORE_THRESH = 0.1
_R = 32
_C = 128


def _sup_tile(rb_ref, colaux_ref, cj):
    def ra(d):
        return rb_ref[d]

    def ca(d):
        return colaux_ref[d, pl.ds(cj, 1), :]

    ix = jnp.maximum(jnp.minimum(ra(3), ca(3)) - jnp.maximum(ra(0), ca(0)), 0.0)
    iy = jnp.maximum(jnp.minimum(ra(4), ca(4)) - jnp.maximum(ra(1), ca(1)), 0.0)
    iz = jnp.maximum(jnp.minimum(ra(5), ca(5)) - jnp.maximum(ra(2), ca(2)), 0.0)
    inter = ix * iy * iz
    union = jnp.maximum(ra(6) + ca(6) - inter, 1e-6)
    return (inter > _NMS_THRESH * union).astype(jnp.float32)


def _nms_body(colaux_ref, rowaux_ref, valid_ref, keep_ref, rb_ref):
    keep_ref[...] = valid_ref[...]
    sub = jax.lax.broadcasted_iota(jnp.int32, (_C, _C), 0)
    lanesq = jax.lax.broadcasted_iota(jnp.int32, (_C, _C), 1)

    def block(bi, carry):
        r0 = bi * _C
        for d in range(7):
            rb_ref[d] = jnp.broadcast_to(
                rowaux_ref[d, pl.ds(r0, _C), :], (_C, _C))
        s_bb = _sup_tile(rb_ref, colaux_ref, bi)
        s_bb = jnp.where(lanesq > sub, s_bb, 0.0)
        kb = keep_ref[pl.ds(bi, 1), :]

        def w_cond(c):
            return c[1]

        def w_body(c):
            x, _ = c
            sup = jnp.dot(x, s_bb, preferred_element_type=jnp.float32)
            nx = jnp.where(sup > 0.0, 0.0, kb)
            return nx, jnp.any(nx != x)

        x, _ = jax.lax.while_loop(w_cond, w_body, (kb, True))
        keep_ref[pl.ds(bi, 1), :] = x

        def chunk8(q, carry2):
            for t in range(16):
                cj = q * 16 + t
                s_bc = _sup_tile(rb_ref, colaux_ref, cj)
                sup = jnp.dot(x, s_bc, preferred_element_type=jnp.float32)
                live = (cj > bi).astype(jnp.float32)
                krow = keep_ref[pl.ds(cj, 1), :]
                keep_ref[pl.ds(cj, 1), :] = jnp.where(
                    sup * live > 0.0, 0.0, krow)
            return carry2

        jax.lax.fori_loop((bi + 1) // 16, _R // 16, chunk8, 0)
        return carry

    jax.lax.fori_loop(0, _R, block, 0)


_NW = 32
_BPW = _PRE // _NW


def _sc_gather(table16, idx):
    mesh = plsc.VectorSubcoreMesh(core_axis_name="c", subcore_axis_name="s")

    @functools.partial(
        pl.kernel, mesh=mesh,
        out_type=jax.ShapeDtypeStruct((_PRE, 128), jnp.float32),
        scratch_types=[
            pltpu.VMEM((_BPW,), jnp.int32),
            pltpu.VMEM((_BPW, 128), jnp.float32),
            pltpu.SemaphoreType.DMA,
        ],
    )
    def k(table_hbm, idx_hbm, out_hbm, idx_v, rows_v, sem):
        wid = jax.lax.axis_index("s") * 2 + jax.lax.axis_index("c")
        base = wid * _BPW
        pltpu.sync_copy(idx_hbm.at[pl.ds(base, _BPW)], idx_v)
        pltpu.async_copy(table_hbm.at[idx_v], rows_v, sem).wait()
        pltpu.sync_copy(rows_v, out_hbm.at[pl.ds(base, _BPW)])

    return k(table16, idx)


def kernel(box_scores, box_preds):
    _, idx = jax.lax.top_k(box_scores, _PRE)
    table = jnp.pad(
        jnp.concatenate([box_preds, box_scores[:, None]], axis=1),
        ((0, 0), (0, 120)))
    g = _sc_gather(table, idx)
    s = g[:, 7]
    b = g[:, :7]
    c = b[:, :3]
    d = jnp.abs(b[:, 3:6])
    mn = c - d * 0.5
    mx = c + d * 0.5
    vol = (d[:, 0] * d[:, 1] * d[:, 2])[:, None]
    cat = jnp.concatenate([mn, mx, vol], axis=1).T
    colaux = cat.reshape(7, _R, _C)
    rowaux = cat.reshape(7, _PRE, 1)
    valid = (s > _SCORE_THRESH).astype(jnp.float32).reshape(_R, _C)
    keep_f = pl.pallas_call(
        _nms_body,
        out_shape=jax.ShapeDtypeStruct((_R, _C), jnp.float32),
        scratch_shapes=[pltpu.VMEM((7, _C, _C), jnp.float32)],
    )(colaux, rowaux, valid)
    keep = keep_f.reshape(_PRE) > 0.5
    kept_masked = jnp.where(keep, s, -1.0)
    _, order = jax.lax.top_k(kept_masked, _POST)
    sel_valid = keep[order]
    sel_scores = jnp.where(sel_valid, s[order], 0.0)
    sel_idx = jnp.where(sel_valid, idx[order], -1)
    return sel_scores, sel_idx


# speedup vs baseline: 1.0458x; 1.0081x over previous
import functools

import jax
import jax.numpy as jnp
from jax.experimental import pallas as pl
from jax.experimental.pallas import tpu as pltpu
from jax.experimental.pallas import tpu_sc as plsc

_N = 20000
_PRE = 4096
_POST = 500
_NMS_THRESH = 0.1
_SCORE_THRESH = 0.1
_R = 32
_C = 128


def _sup_tile(rb_ref, colaux_ref, cj):
    def ra(d):
        return rb_ref[d]

    def ca(d):
        return colaux_ref[d, pl.ds(cj, 1), :]

    ix = jnp.maximum(jnp.minimum(ra(3), ca(3)) - jnp.maximum(ra(0), ca(0)), 0.0)
    iy = jnp.maximum(jnp.minimum(ra(4), ca(4)) - jnp.maximum(ra(1), ca(1)), 0.0)
    iz = jnp.maximum(jnp.minimum(ra(5), ca(5)) - jnp.maximum(ra(2), ca(2)), 0.0)
    inter = ix * iy * iz
    union = jnp.maximum(ra(6) + ca(6) - inter, 1e-6)
    return (inter > _NMS_THRESH * union).astype(jnp.float32)


def _nms_body(colaux_ref, rowaux_ref, valid_ref, keep_ref, rb_ref):
    keep_ref[...] = valid_ref[...]
    sub = jax.lax.broadcasted_iota(jnp.int32, (_C, _C), 0)
    lanesq = jax.lax.broadcasted_iota(jnp.int32, (_C, _C), 1)

    def block(bi, carry):
        r0 = bi * _C
        for d in range(7):
            rb_ref[d] = jnp.broadcast_to(
                rowaux_ref[d, pl.ds(r0, _C), :], (_C, _C))
        s_bb = _sup_tile(rb_ref, colaux_ref, bi)
        s_bb = jnp.where(lanesq > sub, s_bb, 0.0)
        kb = keep_ref[pl.ds(bi, 1), :]

        def w_cond(c):
            return c[1]

        def w_body(c):
            x, _ = c
            sup = jnp.dot(x, s_bb, preferred_element_type=jnp.float32)
            nx = jnp.where(sup > 0.0, 0.0, kb)
            return nx, jnp.any(nx != x)

        x, _ = jax.lax.while_loop(w_cond, w_body, (kb, True))
        keep_ref[pl.ds(bi, 1), :] = x

        def chunk8(q, carry2):
            for t in range(8):
                cj = q * 8 + t
                s_bc = _sup_tile(rb_ref, colaux_ref, cj)
                sup = jnp.dot(x, s_bc, preferred_element_type=jnp.float32)
                live = (cj > bi).astype(jnp.float32)
                krow = keep_ref[pl.ds(cj, 1), :]
                keep_ref[pl.ds(cj, 1), :] = jnp.where(
                    sup * live > 0.0, 0.0, krow)
            return carry2

        jax.lax.fori_loop((bi + 1) // 8, _R // 8, chunk8, 0)
        return carry

    jax.lax.fori_loop(0, _R, block, 0)


_NW = 32
_BPW = _PRE // _NW


def _sc_gather(table16, idx):
    mesh = plsc.VectorSubcoreMesh(core_axis_name="c", subcore_axis_name="s")

    @functools.partial(
        pl.kernel, mesh=mesh,
        out_type=jax.ShapeDtypeStruct((_PRE, 128), jnp.float32),
        scratch_types=[
            pltpu.VMEM((_BPW,), jnp.int32),
            pltpu.VMEM((_BPW, 128), jnp.float32),
            pltpu.SemaphoreType.DMA,
        ],
    )
    def k(table_hbm, idx_hbm, out_hbm, idx_v, rows_v, sem):
        wid = jax.lax.axis_index("s") * 2 + jax.lax.axis_index("c")
        base = wid * _BPW
        pltpu.sync_copy(idx_hbm.at[pl.ds(base, _BPW)], idx_v)
        pltpu.async_copy(table_hbm.at[idx_v], rows_v, sem).wait()
        pltpu.sync_copy(rows_v, out_hbm.at[pl.ds(base, _BPW)])

    return k(table16, idx)


def kernel(box_scores, box_preds):
    _, idx = jax.lax.top_k(box_scores, _PRE)
    table = jnp.pad(
        jnp.concatenate([box_preds, box_scores[:, None]], axis=1),
        ((0, 0), (0, 120)))
    g = _sc_gather(table, idx)
    s = g[:, 7]
    b = g[:, :7]
    c = b[:, :3]
    d = jnp.abs(b[:, 3:6])
    mn = c - d * 0.5
    mx = c + d * 0.5
    vol = (d[:, 0] * d[:, 1] * d[:, 2])[:, None]
    cat = jnp.concatenate([mn, mx, vol], axis=1).T
    colaux = cat.reshape(7, _R, _C)
    rowaux = cat.reshape(7, _PRE, 1)
    valid = (s > _SCORE_THRESH).astype(jnp.float32).reshape(_R, _C)
    keep_f = pl.pallas_call(
        _nms_body,
        out_shape=jax.ShapeDtypeStruct((_R, _C), jnp.float32),
        scratch_shapes=[pltpu.VMEM((7, _C, _C), jnp.float32)],
    )(colaux, rowaux, valid)
    keep = keep_f.reshape(_PRE) > 0.5
    kept_masked = jnp.where(keep, s, -1.0)
    _, order = jax.lax.top_k(kept_masked, _POST)
    sel_valid = keep[order]
    sel_scores = jnp.where(sel_valid, s[order], 0.0)
    sel_idx = jnp.where(sel_valid, idx[order], -1)
    return sel_scores, sel_idx
